# baseline (speedup 1.0000x reference)
import jax
import jax.numpy as jnp
from jax import lax
from jax.experimental import pallas as pl
from jax.experimental.pallas import tpu as pltpu
from jax.experimental.pallas import tpu_sc as plsc

_MASK = 1_000_000
_NIDS = 100_000
_NSUB = 16
_NCORE = 2
_SHARE = _NIDS // _NSUB
_WIN = 6_272
_CH = 128
_N_CH = 49
_NW = 1 << 18
_HW = _NW // 2
_WSL = _HW // _NSUB
_WSL2 = _WSL // 2


def _scatter_body(ids_hbm, out_hbm, half, idx_v, sidx, sval, zbuf,
                  stage_a, stage_b, sem_i, sem_z, sem_s):
    c = lax.axis_index("c")
    s = lax.axis_index("s")
    wbase = c * _HW
    skew = (s * _SHARE) & 7
    start = pl.multiple_of(s * _SHARE - skew, 8)

    idx_dma = pltpu.async_copy(ids_hbm.at[pl.ds(start, _WIN - 16)],
                               idx_v.at[pl.ds(0, _WIN - 16)], sem_i)

    zvec = jnp.zeros((16,), jnp.int32)

    @pl.loop(0, _WSL // 16)
    def _(i):
        zbuf[pl.ds(i * 16, 16)] = zvec

    zero_dma = pltpu.async_copy(zbuf, half.at[pl.ds(s * _WSL, _WSL)], sem_z)

    idx_dma.wait()

    iota16 = lax.iota(jnp.int32, 16)

    def transform(jc, k, lo_mask, hi_mask):
        v = idx_v[pl.ds(jc * _CH + k * 16, 16)]
        inr = (v >= 0) & (((v >> 17) & 1) == c)
        if lo_mask or hi_mask:
            pos = jc * _CH + k * 16 + iota16
            if lo_mask:
                inr &= pos >= skew
            if hi_mask:
                inr &= pos < skew + _SHARE
        val = jnp.where(inr, 1 << (((v >> 18) & 3) << 3), 0)
        sidx[jc, pl.ds(k * 16, 16)] = v & (_HW - 1)
        sval[jc, pl.ds(k * 16, 16)] = val

    for k in range(_CH // 16):
        transform(0, k, True, False)

    @pl.loop(_CH // 16, (_N_CH - 1) * (_CH // 16))
    def _(t):
        transform(t >> 3, t & 7, False, False)

    for k in range(_CH // 16):
        transform(_N_CH - 1, k, False, True)

    zero_dma.wait()
    plsc.subcore_barrier()

    @pl.loop(0, _N_CH)
    def _(jc):
        pltpu.async_copy(sval.at[jc], half.at[sidx.at[jc]], sem_s, add=True)

    @pl.loop(0, _N_CH)
    def _(jc):
        pltpu.make_async_copy(sval.at[jc], half.at[sidx.at[jc]], sem_s).wait()

    plsc.subcore_barrier()

    d_a = pltpu.async_copy(half.at[pl.ds(s * _WSL, _WSL2)], stage_a, sem_z)
    d_b = pltpu.async_copy(half.at[pl.ds(s * _WSL + _WSL2, _WSL2)], stage_b,
                           sem_i)
    d_a.wait()
    o_a = pltpu.async_copy(stage_a, out_hbm.at[pl.ds(wbase + s * _WSL, _WSL2)],
                           sem_z)
    d_b.wait()
    o_b = pltpu.async_copy(stage_b,
                           out_hbm.at[pl.ds(wbase + s * _WSL + _WSL2, _WSL2)],
                           sem_i)
    o_a.wait()
    o_b.wait()


def kernel(in_ids, size_tensor):
    assert size_tensor.shape[0] == _MASK and in_ids.shape[0] == _NIDS
    ids = in_ids.astype(jnp.int32)

    mesh = plsc.VectorSubcoreMesh(core_axis_name="c", subcore_axis_name="s",
                                  num_cores=_NCORE, num_subcores=_NSUB)
    run = pl.kernel(
        _scatter_body,
        out_type=jax.ShapeDtypeStruct((_NW,), jnp.int32),
        mesh=mesh,
        compiler_params=pltpu.CompilerParams(needs_layout_passes=False),
        scratch_types=[
            pltpu.VMEM_SHARED((_HW,), jnp.int32),
            pltpu.VMEM((_WIN,), jnp.int32),
            pltpu.VMEM((_N_CH, _CH), jnp.int32),
            pltpu.VMEM((_N_CH, _CH), jnp.int32),
            pltpu.VMEM((_WSL,), jnp.int32),
            pltpu.VMEM((_WSL2,), jnp.int32),
            pltpu.VMEM((_WSL2,), jnp.int32),
            pltpu.SemaphoreType.DMA,
            pltpu.SemaphoreType.DMA,
            pltpu.SemaphoreType.DMA,
        ],
    )
    w = run(ids)
    planes = [((w >> (8 * p)) & 0xFF) != 0 for p in range(4)]
    planes[3] = planes[3][: _MASK - 3 * _NW]
    return jnp.concatenate(planes)


# speedup vs baseline: 1.0133x; 1.0133x over previous
import jax
import jax.numpy as jnp
from jax import lax
from jax.experimental import pallas as pl
from jax.experimental.pallas import tpu as pltpu
from jax.experimental.pallas import tpu_sc as plsc

_MASK = 1_000_000
_NIDS = 100_000
_NSUB = 16
_NCORE = 2
_SHARE = _NIDS // _NSUB
_WIN = 6_272
_CH = 128
_N_CH = 49
_NW = 1 << 18
_HW = _NW // 2
_WSL = _HW // _NSUB
_WSL2 = _WSL // 2


def _scatter_body(ids_hbm, out_hbm, half, idx_v, sidx, sval, zbuf,
                  stage_a, stage_b, sem_i, sem_z, sem_s):
    c = lax.axis_index("c")
    s = lax.axis_index("s")
    wbase = c * _HW
    skew = (s * _SHARE) & 7
    start = pl.multiple_of(s * _SHARE - skew, 8)

    idx_dma = pltpu.async_copy(ids_hbm.at[pl.ds(start, _WIN - 16)],
                               idx_v.at[pl.ds(0, _WIN - 16)], sem_i)

    zvec = jnp.zeros((16,), jnp.int32)

    @pl.loop(0, _WSL // 16)
    def _(i):
        zbuf[pl.ds(i * 16, 16)] = zvec

    zero_dma = pltpu.async_copy(zbuf, half.at[pl.ds(s * _WSL, _WSL)], sem_z)

    idx_dma.wait()

    iota16 = lax.iota(jnp.int32, 16)

    def transform(jc, k, lo_mask, hi_mask):
        v = idx_v[pl.ds(jc * _CH + k * 16, 16)]
        inr = (v >= 0) & (((v >> 17) & 1) == c)
        if lo_mask or hi_mask:
            pos = jc * _CH + k * 16 + iota16
            if lo_mask:
                inr &= pos >= skew
            if hi_mask:
                inr &= pos < skew + _SHARE
        val = jnp.where(inr, 1 << (((v >> 18) & 3) << 3), 0)
        sidx[jc, pl.ds(k * 16, 16)] = v & (_HW - 1)
        sval[jc, pl.ds(k * 16, 16)] = val

    for k in range(_CH // 16):
        transform(0, k, True, False)

    @pl.loop(1, _N_CH - 1)
    def _(jc):
        for k in range(_CH // 16):
            transform(jc, k, False, False)

    for k in range(_CH // 16):
        transform(_N_CH - 1, k, False, True)

    zero_dma.wait()
    plsc.subcore_barrier()

    @pl.loop(0, _N_CH)
    def _(jc):
        pltpu.async_copy(sval.at[jc], half.at[sidx.at[jc]], sem_s, add=True)

    @pl.loop(0, _N_CH)
    def _(jc):
        pltpu.make_async_copy(sval.at[jc], half.at[sidx.at[jc]], sem_s).wait()

    plsc.subcore_barrier()

    d_a = pltpu.async_copy(half.at[pl.ds(s * _WSL, _WSL2)], stage_a, sem_z)
    d_b = pltpu.async_copy(half.at[pl.ds(s * _WSL + _WSL2, _WSL2)], stage_b,
                           sem_i)
    d_a.wait()
    o_a = pltpu.async_copy(stage_a, out_hbm.at[pl.ds(wbase + s * _WSL, _WSL2)],
                           sem_z)
    d_b.wait()
    o_b = pltpu.async_copy(stage_b,
                           out_hbm.at[pl.ds(wbase + s * _WSL + _WSL2, _WSL2)],
                           sem_i)
    o_a.wait()
    o_b.wait()


def kernel(in_ids, size_tensor):
    assert size_tensor.shape[0] == _MASK and in_ids.shape[0] == _NIDS
    ids = in_ids.astype(jnp.int32)

    mesh = plsc.VectorSubcoreMesh(core_axis_name="c", subcore_axis_name="s",
                                  num_cores=_NCORE, num_subcores=_NSUB)
    run = pl.kernel(
        _scatter_body,
        out_type=jax.ShapeDtypeStruct((_NW,), jnp.int32),
        mesh=mesh,
        compiler_params=pltpu.CompilerParams(needs_layout_passes=False,
                                             disable_semaphore_checks=True,
                                             skip_device_barrier=True),
        scratch_types=[
            pltpu.VMEM_SHARED((_HW,), jnp.int32),
            pltpu.VMEM((_WIN,), jnp.int32),
            pltpu.VMEM((_N_CH, _CH), jnp.int32),
            pltpu.VMEM((_N_CH, _CH), jnp.int32),
            pltpu.VMEM((_WSL,), jnp.int32),
            pltpu.VMEM((_WSL2,), jnp.int32),
            pltpu.VMEM((_WSL2,), jnp.int32),
            pltpu.SemaphoreType.DMA,
            pltpu.SemaphoreType.DMA,
            pltpu.SemaphoreType.DMA,
        ],
    )
    w = run(ids)
    planes = [((w >> (8 * p)) & 0xFF) != 0 for p in range(4)]
    planes[3] = planes[3][: _MASK - 3 * _NW]
    return jnp.concatenate(planes)


# speedup vs baseline: 1.0306x; 1.0170x over previous
import jax
import jax.numpy as jnp
from jax import lax
from jax.experimental import pallas as pl
from jax.experimental.pallas import tpu as pltpu
from jax.experimental.pallas import tpu_sc as plsc

_MASK = 1_000_000
_NIDS = 100_000
_NSUB = 16
_NCORE = 2
_SHARE = _NIDS // _NSUB
_WIN = 6_272
_CH = 128
_N_CH = 49
_NW = 1 << 18
_HW = _NW // 2
_WSL = _HW // _NSUB
_WSL2 = _WSL // 2


def _scatter_body(ids_hbm, out_hbm, half, idx_v, sidx, sval, zbuf,
                  stage_a, stage_b, sem_i, sem_z, sem_s):
    c = lax.axis_index("c")
    s = lax.axis_index("s")
    wbase = c * _HW
    skew = (s * _SHARE) & 7
    start = pl.multiple_of(s * _SHARE - skew, 8)

    idx_dma = pltpu.async_copy(ids_hbm.at[pl.ds(start, _WIN - 16)],
                               idx_v.at[pl.ds(0, _WIN - 16)], sem_i)

    zvec = jnp.zeros((16,), jnp.int32)

    @pl.loop(0, _WSL // 16)
    def _(i):
        zbuf[pl.ds(i * 16, 16)] = zvec

    zero_dma = pltpu.async_copy(zbuf, half.at[pl.ds(s * _WSL, _WSL)], sem_z)
    zero_dma.wait()
    plsc.subcore_barrier()
    idx_dma.wait()

    iota16 = lax.iota(jnp.int32, 16)

    def fire(jc):
        pltpu.async_copy(sval.at[jc], half.at[sidx.at[jc]], sem_s, add=True)

    def transform(jc, k, lo_mask, hi_mask):
        v = idx_v[pl.ds(jc * _CH + k * 16, 16)]
        inr = (v >= 0) & (((v >> 17) & 1) == c)
        if lo_mask or hi_mask:
            pos = jc * _CH + k * 16 + iota16
            if lo_mask:
                inr &= pos >= skew
            if hi_mask:
                inr &= pos < skew + _SHARE
        val = jnp.where(inr, 1 << (((v >> 18) & 3) << 3), 0)
        sidx[jc, pl.ds(k * 16, 16)] = v & (_HW - 1)
        sval[jc, pl.ds(k * 16, 16)] = val

    for k in range(_CH // 16):
        transform(0, k, True, False)
    fire(0)

    @pl.loop(1, _N_CH - 1)
    def _(jc):
        for k in range(_CH // 16):
            transform(jc, k, False, False)
        fire(jc)

    for k in range(_CH // 16):
        transform(_N_CH - 1, k, False, True)
    fire(_N_CH - 1)

    @pl.loop(0, _N_CH)
    def _(jc):
        pltpu.make_async_copy(sval.at[jc], half.at[sidx.at[jc]], sem_s).wait()

    plsc.subcore_barrier()

    d_a = pltpu.async_copy(half.at[pl.ds(s * _WSL, _WSL2)], stage_a, sem_z)
    d_b = pltpu.async_copy(half.at[pl.ds(s * _WSL + _WSL2, _WSL2)], stage_b,
                           sem_i)
    d_a.wait()
    o_a = pltpu.async_copy(stage_a, out_hbm.at[pl.ds(wbase + s * _WSL, _WSL2)],
                           sem_z)
    d_b.wait()
    o_b = pltpu.async_copy(stage_b,
                           out_hbm.at[pl.ds(wbase + s * _WSL + _WSL2, _WSL2)],
                           sem_i)
    o_a.wait()
    o_b.wait()


def kernel(in_ids, size_tensor):
    assert size_tensor.shape[0] == _MASK and in_ids.shape[0] == _NIDS
    ids = in_ids.astype(jnp.int32)

    mesh = plsc.VectorSubcoreMesh(core_axis_name="c", subcore_axis_name="s",
                                  num_cores=_NCORE, num_subcores=_NSUB)
    run = pl.kernel(
        _scatter_body,
        out_type=jax.ShapeDtypeStruct((_NW,), jnp.int32),
        mesh=mesh,
        compiler_params=pltpu.CompilerParams(needs_layout_passes=False),
        scratch_types=[
            pltpu.VMEM_SHARED((_HW,), jnp.int32),
            pltpu.VMEM((_WIN,), jnp.int32),
            pltpu.VMEM((_N_CH, _CH), jnp.int32),
            pltpu.VMEM((_N_CH, _CH), jnp.int32),
            pltpu.VMEM((_WSL,), jnp.int32),
            pltpu.VMEM((_WSL2,), jnp.int32),
            pltpu.VMEM((_WSL2,), jnp.int32),
            pltpu.SemaphoreType.DMA,
            pltpu.SemaphoreType.DMA,
            pltpu.SemaphoreType.DMA,
        ],
    )
    w = run(ids)
    planes = [((w >> (8 * p)) & 0xFF) != 0 for p in range(4)]
    planes[3] = planes[3][: _MASK - 3 * _NW]
    return jnp.concatenate(planes)


# speedup vs baseline: 1.0714x; 1.0396x over previous
import jax
import jax.numpy as jnp
from jax import lax
from jax.experimental import pallas as pl
from jax.experimental.pallas import tpu as pltpu
from jax.experimental.pallas import tpu_sc as plsc

_MASK = 1_000_000
_NIDS = 100_000
_NSUB = 16
_NCORE = 2
_SHARE = _NIDS // _NSUB
_WIN = 6_272
_CH = 128
_N_CH = 49
_NW = 1 << 17
_HW = _NW // 2
_WSL = _HW // _NSUB
_WSL2 = _WSL // 2


def _scatter_body(ids_hbm, out_hbm, half, idx_v, sidx, sval, zbuf,
                  stage_a, stage_b, sem_i, sem_z, sem_s):
    c = lax.axis_index("c")
    s = lax.axis_index("s")
    wbase = c * _HW
    skew = (s * _SHARE) & 7
    start = pl.multiple_of(s * _SHARE - skew, 8)

    idx_dma = pltpu.async_copy(ids_hbm.at[pl.ds(start, _WIN - 16)],
                               idx_v.at[pl.ds(0, _WIN - 16)], sem_i)

    zvec = jnp.zeros((16,), jnp.int32)

    @pl.loop(0, _WSL // 16)
    def _(i):
        zbuf[pl.ds(i * 16, 16)] = zvec

    zero_dma = pltpu.async_copy(zbuf, half.at[pl.ds(s * _WSL, _WSL)], sem_z)
    zero_dma.wait()
    plsc.subcore_barrier()
    idx_dma.wait()

    iota16 = lax.iota(jnp.int32, 16)

    def fire(jc):
        pltpu.async_copy(sval.at[jc], half.at[sidx.at[jc]], sem_s, add=True)

    def transform(jc, k, lo_mask, hi_mask):
        v = idx_v[pl.ds(jc * _CH + k * 16, 16)]
        inr = (v >= 0) & (((v >> 16) & 1) == c)
        if lo_mask or hi_mask:
            pos = jc * _CH + k * 16 + iota16
            if lo_mask:
                inr &= pos >= skew
            if hi_mask:
                inr &= pos < skew + _SHARE
        val = jnp.where(inr, 1 << (((v >> 17) & 7) << 2), 0)
        sidx[jc, pl.ds(k * 16, 16)] = v & (_HW - 1)
        sval[jc, pl.ds(k * 16, 16)] = val

    for k in range(_CH // 16):
        transform(0, k, True, False)
    fire(0)

    @pl.loop(1, _N_CH - 1)
    def _(jc):
        for k in range(_CH // 16):
            transform(jc, k, False, False)
        fire(jc)

    for k in range(_CH // 16):
        transform(_N_CH - 1, k, False, True)
    fire(_N_CH - 1)

    @pl.loop(0, _N_CH)
    def _(jc):
        pltpu.make_async_copy(sval.at[jc], half.at[sidx.at[jc]], sem_s).wait()

    plsc.subcore_barrier()

    d_a = pltpu.async_copy(half.at[pl.ds(s * _WSL, _WSL2)], stage_a, sem_z)
    d_b = pltpu.async_copy(half.at[pl.ds(s * _WSL + _WSL2, _WSL2)], stage_b,
                           sem_i)
    d_a.wait()
    o_a = pltpu.async_copy(stage_a, out_hbm.at[pl.ds(wbase + s * _WSL, _WSL2)],
                           sem_z)
    d_b.wait()
    o_b = pltpu.async_copy(stage_b,
                           out_hbm.at[pl.ds(wbase + s * _WSL + _WSL2, _WSL2)],
                           sem_i)
    o_a.wait()
    o_b.wait()


def kernel(in_ids, size_tensor):
    assert size_tensor.shape[0] == _MASK and in_ids.shape[0] == _NIDS
    ids = in_ids.astype(jnp.int32)

    mesh = plsc.VectorSubcoreMesh(core_axis_name="c", subcore_axis_name="s",
                                  num_cores=_NCORE, num_subcores=_NSUB)
    run = pl.kernel(
        _scatter_body,
        out_type=jax.ShapeDtypeStruct((_NW,), jnp.int32),
        mesh=mesh,
        compiler_params=pltpu.CompilerParams(needs_layout_passes=False),
        scratch_types=[
            pltpu.VMEM_SHARED((_HW,), jnp.int32),
            pltpu.VMEM((_WIN,), jnp.int32),
            pltpu.VMEM((_N_CH, _CH), jnp.int32),
            pltpu.VMEM((_N_CH, _CH), jnp.int32),
            pltpu.VMEM((_WSL,), jnp.int32),
            pltpu.VMEM((_WSL2,), jnp.int32),
            pltpu.VMEM((_WSL2,), jnp.int32),
            pltpu.SemaphoreType.DMA,
            pltpu.SemaphoreType.DMA,
            pltpu.SemaphoreType.DMA,
        ],
    )
    w = run(ids)
    planes = [((w >> (4 * p)) & 0xF) != 0 for p in range(8)]
    planes[7] = planes[7][: _MASK - 7 * _NW]
    return jnp.concatenate(planes)
